# Initial kernel scaffold; baseline (speedup 1.0000x reference)
import jax
import jax.numpy as jnp
from jax.experimental import pallas as pl


def kernel(input, triple, rel_table, W, conv_w, conv_b, bn1_gamma, bn1_beta, bn2_gamma, bn2_beta, fc_w):
    raise NotImplementedError("write your pallas kernel here")


# speedup vs baseline: 5.1112x; 5.1112x over previous
import dataclasses
import functools

import jax
import jax.numpy as jnp
from jax import lax
from jax.experimental import pallas as pl
from jax.experimental.pallas import tpu as pltpu
from jax.experimental.pallas import tpu_sc as plsc

_CHUNK = 128
_BE = 2000
_BN = 2000


def _matmul(x, w):
    n, d = x.shape

    def body(x_ref, w_ref, o_ref):
        o_ref[...] = jnp.dot(x_ref[...], w_ref[...],
                             preferred_element_type=jnp.float32)

    return pl.pallas_call(
        body,
        grid=(n // _BN,),
        in_specs=[
            pl.BlockSpec((_BN, d), lambda i: (i, 0)),
            pl.BlockSpec((d, d), lambda i: (0, 0)),
        ],
        out_specs=pl.BlockSpec((_BN, d), lambda i: (i, 0)),
        out_shape=jax.ShapeDtypeStruct((n, d), jnp.float32),
    )(x, w)


def _sc_gather(table_h, table_r, hidx, ridx, cidx):
    e = hidx.shape[0]
    d = table_h.shape[1]
    nchunk = e // _CHUNK
    info = plsc.get_sparse_core_info()
    nw = info.num_cores * info.num_subcores
    iters = (nchunk + nw - 1) // nw
    mesh = plsc.VectorSubcoreMesh(core_axis_name="c", subcore_axis_name="s")
    out_t = jax.ShapeDtypeStruct((e, d), jnp.float32)

    @functools.partial(
        pl.kernel, mesh=mesh,
        out_type=[out_t, out_t, out_t],
        scratch_types=[
            pltpu.VMEM((_CHUNK,), jnp.int32),
            pltpu.VMEM((_CHUNK,), jnp.int32),
            pltpu.VMEM((_CHUNK,), jnp.int32),
            pltpu.VMEM((_CHUNK, d), jnp.float32),
            pltpu.VMEM((_CHUNK, d), jnp.float32),
            pltpu.VMEM((_CHUNK, d), jnp.float32),
            pltpu.SemaphoreType.DMA,
            pltpu.SemaphoreType.DMA,
            pltpu.SemaphoreType.DMA,
        ],
    )
    def k(th_hbm, tr_hbm, hi_hbm, ri_hbm, ci_hbm, ho_hbm, ro_hbm, to_hbm,
          hi_v, ri_v, ci_v, hb, rb, tb, s0, s1, s2):
        w = lax.axis_index("s") * info.num_cores + lax.axis_index("c")

        @pl.loop(0, iters)
        def _(jj):
            j = w + jj * nw

            @pl.when(j < nchunk)
            def _():
                base = j * _CHUNK
                pltpu.sync_copy(hi_hbm.at[pl.ds(base, _CHUNK)], hi_v)
                pltpu.sync_copy(ri_hbm.at[pl.ds(base, _CHUNK)], ri_v)
                pltpu.sync_copy(ci_hbm.at[pl.ds(base, _CHUNK)], ci_v)
                c0 = pltpu.async_copy(th_hbm.at[hi_v], hb, s0)
                c1 = pltpu.async_copy(tr_hbm.at[ri_v], rb, s1)
                c2 = pltpu.async_copy(th_hbm.at[ci_v], tb, s2)
                c0.wait()
                c1.wait()
                c2.wait()
                pltpu.sync_copy(hb, ho_hbm.at[pl.ds(base, _CHUNK)])
                pltpu.sync_copy(rb, ro_hbm.at[pl.ds(base, _CHUNK)])
                pltpu.sync_copy(tb, to_hbm.at[pl.ds(base, _CHUNK)])

    return k(table_h, table_r, hidx, ridx, cidx)


def _conv_channels(hs, rs, ts, cw_ref):
    outs = []
    for c in range(4):
        acc = None
        for ki in range(3):
            term = (cw_ref[c * 9 + ki * 3 + 0] * hs[ki]
                    + cw_ref[c * 9 + ki * 3 + 1] * rs[ki]
                    + cw_ref[c * 9 + ki * 3 + 2] * ts[ki])
            acc = term if acc is None else acc + term
        outs.append(acc)
    return outs


def _stats_pass(H, R, T, cwf):
    e, d = H.shape
    nb = e // _BE

    def body(cw_ref, h_ref, r_ref, t_ref, o_ref):
        i = pl.program_id(0)

        @pl.when(i == 0)
        def _():
            o_ref[...] = jnp.zeros_like(o_ref)

        h = h_ref[...]
        r = r_ref[...]
        t = t_ref[...]
        vals = [
            jnp.sum(h) + jnp.sum(r) + jnp.sum(t),
            jnp.sum(h * h) + jnp.sum(r * r) + jnp.sum(t * t),
        ]
        hs = [h[:, ki:ki + d - 2] for ki in range(3)]
        rs = [r[:, ki:ki + d - 2] for ki in range(3)]
        ts = [t[:, ki:ki + d - 2] for ki in range(3)]
        for L in _conv_channels(hs, rs, ts, cw_ref):
            vals.append(jnp.sum(L))
            vals.append(jnp.sum(L * L))
        lane = lax.broadcasted_iota(jnp.int32, (1, 128), 1)
        p = jnp.zeros((1, 128), jnp.float32)
        for k, v in enumerate(vals):
            p = p + jnp.where(lane == k, v, 0.0)
        o_ref[...] += p

    return pl.pallas_call(
        body,
        grid=(nb,),
        in_specs=[
            pl.BlockSpec(memory_space=pltpu.SMEM),
            pl.BlockSpec((_BE, d), lambda i: (i, 0)),
            pl.BlockSpec((_BE, d), lambda i: (i, 0)),
            pl.BlockSpec((_BE, d), lambda i: (i, 0)),
        ],
        out_specs=pl.BlockSpec((1, 128), lambda i: (0, 0)),
        out_shape=jax.ShapeDtypeStruct((1, 128), jnp.float32),
    )(cwf, H, R, T)


def _score_pass(H, R, T, cwf, scal):
    e, d = H.shape
    nb = e // _BE
    dc = d - 2

    def body(cw_ref, h_ref, r_ref, t_ref, scal_ref, o_ref):
        h = h_ref[...]
        r = r_ref[...]
        t = t_ref[...]
        hs = [h[:, ki:ki + dc] for ki in range(3)]
        rs = [r[:, ki:ki + dc] for ki in range(3)]
        ts = [t[:, ki:ki + dc] for ki in range(3)]
        tot = jnp.zeros((h.shape[0], dc), jnp.float32)
        for c, L in enumerate(_conv_channels(hs, rs, ts, cw_ref)):
            z = L * scal_ref[c:c + 1, :dc] + scal_ref[4 + c:5 + c, :dc]
            tot = tot + jnp.maximum(z, 0.0) * scal_ref[8 + c:9 + c, :dc]
        ev = jnp.sum(tot, axis=1, keepdims=True)
        ev = jnp.where(ev >= 0.0, ev, 0.01 * ev)
        o_ref[...] = jnp.exp(ev)

    return pl.pallas_call(
        body,
        grid=(nb,),
        in_specs=[
            pl.BlockSpec(memory_space=pltpu.SMEM),
            pl.BlockSpec((_BE, d), lambda i: (i, 0)),
            pl.BlockSpec((_BE, d), lambda i: (i, 0)),
            pl.BlockSpec((_BE, d), lambda i: (i, 0)),
            pl.BlockSpec((16, 128), lambda i: (0, 0)),
        ],
        out_specs=pl.BlockSpec((_BE, 1), lambda i: (i, 0)),
        out_shape=jax.ShapeDtypeStruct((e, 1), jnp.float32),
    )(cwf, H, R, T, scal)


def _sc_aggregate(ex, row, col, input_):
    e = ex.shape[0]
    n, d = input_.shape
    nchunk = e // _CHUNK
    info = plsc.get_sparse_core_info()
    ncores, nsub = info.num_cores, info.num_subcores
    nw = ncores * nsub
    iters = (nchunk + nw - 1) // nw
    rcp = 80
    nrchunk = n // rcp
    iters_z = (nrchunk + nsub - 1) // nsub
    ndr = -(-n // d) + (-(-n // d)) % 8
    mesh = plsc.VectorSubcoreMesh(core_axis_name="c", subcore_axis_name="s")
    cp = pltpu.CompilerParams()
    if "needs_layout_passes" in pltpu.CompilerParams.__dataclass_fields__:
        cp = dataclasses.replace(cp, needs_layout_passes=False)

    @functools.partial(
        pl.kernel, mesh=mesh, compiler_params=cp,
        out_type=[jax.ShapeDtypeStruct((ncores, n, d), jnp.float32),
                  jax.ShapeDtypeStruct((ncores, ndr, d), jnp.float32)],
        scratch_types=[
            pltpu.VMEM((_CHUNK,), jnp.float32),
            pltpu.VMEM((_CHUNK,), jnp.int32),
            pltpu.VMEM((_CHUNK,), jnp.int32),
            pltpu.VMEM((_CHUNK,), jnp.int32),
            pltpu.VMEM((_CHUNK,), jnp.int32),
            pltpu.VMEM((_CHUNK, d), jnp.float32),
            pltpu.VMEM((_CHUNK, d), jnp.float32),
            pltpu.VMEM((_CHUNK, d), jnp.float32),
            pltpu.VMEM_SHARED((n, d), jnp.float32),
            pltpu.VMEM_SHARED((ndr, d), jnp.float32),
            pltpu.SemaphoreType.DMA,
        ],
    )
    def k(ex_hbm, row_hbm, col_hbm, in_hbm, agg_hbm, den_hbm,
          ex_v, row_v, col_v, rowd_v, pos_v, rows_v, val_v, val2_v,
          agg_sh, den_sh, sem):
        cid = lax.axis_index("c")
        sid = lax.axis_index("s")
        w = sid * ncores + cid

        @pl.loop(0, _CHUNK)
        def _(i):
            for kk in range(d // 16):
                val_v[i, pl.ds(kk * 16, 16)] = jnp.zeros((16,), jnp.float32)
                val2_v[i, pl.ds(kk * 16, 16)] = jnp.zeros((16,), jnp.float32)

        @pl.loop(0, iters_z)
        def _(z):
            c = sid + z * nsub

            @pl.when(c < nrchunk)
            def _():
                pltpu.sync_copy(val_v.at[pl.ds(0, rcp)],
                                agg_sh.at[pl.ds(c * rcp, rcp)])

        @pl.when(sid == 0)
        def _():
            pltpu.sync_copy(val_v.at[pl.ds(0, ndr)], den_sh)

        plsc.subcore_barrier()

        @pl.loop(0, iters)
        def _(jj):
            j = w + jj * nw

            @pl.when(j < nchunk)
            def _():
                base = j * _CHUNK
                pltpu.sync_copy(ex_hbm.at[pl.ds(base, _CHUNK)], ex_v)
                pltpu.sync_copy(row_hbm.at[pl.ds(base, _CHUNK)], row_v)
                pltpu.sync_copy(col_hbm.at[pl.ds(base, _CHUNK)], col_v)
                pltpu.async_copy(in_hbm.at[col_v], rows_v, sem).wait()
                iota16 = lax.broadcasted_iota(jnp.int32, (16,), 0)

                @pl.loop(0, _CHUNK // 16)
                def _(g):
                    gbase = g * 16
                    row16 = row_v[pl.ds(gbase, 16)]
                    rowd_v[pl.ds(gbase, 16)] = lax.shift_right_logical(
                        row16, 7)
                    pos_v[pl.ds(gbase, 16)] = lax.rem(row16, 128)

                @pl.loop(0, _CHUNK)
                def _(i):
                    splat_i = jnp.full((16,), i, jnp.int32)
                    sv = plsc.load_gather(ex_v, [splat_i])
                    pv = plsc.load_gather(pos_v, [splat_i])
                    for kk in range(d // 16):
                        val_v[i, pl.ds(kk * 16, 16)] = (
                            rows_v[i, pl.ds(kk * 16, 16)] * sv)
                        val2_v[i, pl.ds(kk * 16, 16)] = jnp.where(
                            iota16 + (kk * 16) == pv, sv,
                            jnp.zeros((16,), jnp.float32))

                pltpu.sync_copy(val_v, agg_sh.at[row_v], add=True)
                pltpu.sync_copy(val2_v, den_sh.at[rowd_v], add=True)

        plsc.subcore_barrier()

        @pl.loop(0, iters_z)
        def _(z):
            c = sid + z * nsub

            @pl.when(c < nrchunk)
            def _():
                pltpu.sync_copy(agg_sh.at[pl.ds(c * rcp, rcp)],
                                agg_hbm.at[cid, pl.ds(c * rcp, rcp)])

        @pl.when(sid == 0)
        def _():
            pltpu.sync_copy(den_sh, den_hbm.at[cid])

    return k(ex, row, col, input_)


def _finalize(input_, agg_pair, den_pair):
    n, d = input_.shape

    def body(x_ref, a_ref, dn_ref, o_ref):
        a = a_ref[0] + a_ref[1]
        den = dn_ref[0] + dn_ref[1]
        agg = jnp.where(den > 0.0, a / den, 0.0)
        out = x_ref[...] + agg
        o_ref[...] = jnp.where(out > 0.0, out, jnp.exp(out) - 1.0)

    return pl.pallas_call(
        body,
        grid=(n // _BN,),
        in_specs=[
            pl.BlockSpec((_BN, d), lambda i: (i, 0)),
            pl.BlockSpec((2, _BN, d), lambda i: (0, i, 0)),
            pl.BlockSpec((2, _BN, 1), lambda i: (0, i, 0)),
        ],
        out_specs=pl.BlockSpec((_BN, d), lambda i: (i, 0)),
        out_shape=jax.ShapeDtypeStruct((n, d), jnp.float32),
    )(input_, agg_pair, den_pair)


def kernel(input, triple, rel_table, W, conv_w, conv_b, bn1_gamma, bn1_beta,
           bn2_gamma, bn2_beta, fc_w):
    n, d = input.shape
    e = triple.shape[0]
    dc = d - 2
    eps = 1e-5

    row = triple[:, 0]
    rel = triple[:, 1]
    col = triple[:, 2]

    input_ = _matmul(input, W)

    H, R, T = _sc_gather(input_, rel_table, row, rel, col)

    cwf = conv_w.reshape(4 * 9)
    stats = _stats_pass(H, R, T, cwf)[0]

    cnt1 = 3.0 * e * d
    mu1 = stats[0] / cnt1
    v1 = stats[1] / cnt1 - mu1 * mu1
    a1 = bn1_gamma[0] / jnp.sqrt(v1 + eps)
    cnt2 = float(e * dc)
    m0 = stats[2:10:2] / cnt2
    v0 = stats[3:10:2] / cnt2 - m0 * m0
    s_c = a1 * bn2_gamma / jnp.sqrt(a1 * a1 * v0 + eps)
    o_c = bn2_beta - s_c * m0

    fc2 = fc_w.reshape(4, dc)
    scal = jnp.zeros((16, 128), jnp.float32)
    scal = scal.at[0:4, :dc].set(jnp.broadcast_to(s_c[:, None], (4, dc)))
    scal = scal.at[4:8, :dc].set(jnp.broadcast_to(o_c[:, None], (4, dc)))
    scal = scal.at[8:12, :dc].set(fc2)

    ex = _score_pass(H, R, T, cwf, scal)[:, 0]

    agg_pair, den_out = _sc_aggregate(ex, row, col, input_)
    den_pair = den_out.reshape(2, -1)[:, :n].reshape(2, n, 1)

    return _finalize(input_, agg_pair, den_pair)
